# Initial kernel scaffold; baseline (speedup 1.0000x reference)
import jax
import jax.numpy as jnp
from jax.experimental import pallas as pl


def kernel(vfe_masked, VFE_full, hyperedge_index, task_importance_mask, neuron_mask, edge_mask, contribution_history, history_idx, low_contrib_count):
    raise NotImplementedError("write your pallas kernel here")


# speedup vs baseline: 338.7721x; 338.7721x over previous
import functools

import jax
import jax.numpy as jnp
from jax import lax
from jax.experimental import pallas as pl
from jax.experimental.pallas import tpu as pltpu
from jax.experimental.pallas import tpu_sc as plsc

N_NEURONS = 100000
N_EDGES = 100000
N_CONN = 3200000
COOLDOWN = 10
DEAD_RATIO = 0.9
VFE_RATIO = 1.5
GROW_CAP = 0.05

NC = 2
NS = 16
NW = NC * NS
L = 16

P = 102400
E_PER_W = P // NW
C_PER_W = N_CONN // NW
CHUNK = 20000
N_CHUNKS = C_PER_W // CHUNK
SLICE = P // NS

_mesh = plsc.VectorSubcoreMesh(
    core_axis_name="c", subcore_axis_name="s", num_cores=NC, num_subcores=NS)


def _wid():
    return lax.axis_index("s") * NC + lax.axis_index("c")


@functools.partial(
    pl.kernel,
    out_type=jax.ShapeDtypeStruct((P,), jnp.int32),
    mesh=_mesh,
    compiler_params=pltpu.CompilerParams(needs_layout_passes=False),
    scratch_types=[
        pltpu.VMEM((E_PER_W,), jnp.float32),
        pltpu.VMEM((L,), jnp.float32),
        pltpu.VMEM((E_PER_W,), jnp.int32),
        pltpu.VMEM((E_PER_W,), jnp.int32),
        pltpu.VMEM((E_PER_W,), jnp.int32),
        pltpu.VMEM((E_PER_W,), jnp.int32),
    ],
)
def _edge_kernel(vfe_hbm, vfull_hbm, task_hbm, em_hbm, lcc_hbm, out_hbm,
                 vfe_v, vfull_v, task_v, em_v, lcc_v, out_v):
    base = pl.multiple_of(_wid() * E_PER_W, 8)
    pltpu.sync_copy(vfe_hbm.at[pl.ds(base, E_PER_W)], vfe_v)
    pltpu.sync_copy(vfull_hbm, vfull_v)
    pltpu.sync_copy(task_hbm.at[pl.ds(base, E_PER_W)], task_v)
    pltpu.sync_copy(em_hbm.at[pl.ds(base, E_PER_W)], em_v)
    pltpu.sync_copy(lcc_hbm.at[pl.ds(base, E_PER_W)], lcc_v)
    vfull = vfull_v[...]

    def body(i, _):
        o = i * L
        vfe = vfe_v[pl.ds(o, L)]
        task = task_v[pl.ds(o, L)]
        em = em_v[pl.ds(o, L)]
        lcc = lcc_v[pl.ds(o, L)]
        contribution = vfe - vfull
        is_low = contribution <= 0.0
        lcc2 = jnp.where(is_low, lcc + 1, 0)
        apop = (lcc2 >= COOLDOWN) & (task == 0) & (em != 0)
        out_v[pl.ds(o, L)] = jnp.where(apop, 0, em)
        return 0

    lax.fori_loop(0, E_PER_W // L, body, 0)
    pltpu.sync_copy(out_v, out_hbm.at[pl.ds(base, E_PER_W)])


@functools.partial(
    pl.kernel,
    out_type=[
        jax.ShapeDtypeStruct((NC * P,), jnp.int32),
        jax.ShapeDtypeStruct((NC * P,), jnp.int32),
    ],
    mesh=_mesh,
    compiler_params=pltpu.CompilerParams(needs_layout_passes=False),
    scratch_types=[
        pltpu.VMEM((CHUNK,), jnp.int32),
        pltpu.VMEM((CHUNK,), jnp.int32),
        pltpu.VMEM((CHUNK,), jnp.int32),
        pltpu.VMEM((CHUNK,), jnp.int32),
        pltpu.VMEM((SLICE,), jnp.int32),
        pltpu.VMEM_SHARED((P,), jnp.int32),
        pltpu.VMEM_SHARED((P,), jnp.int32),
        pltpu.VMEM_SHARED((P,), jnp.int32),
    ],
)
def _scatter_kernel(nids_hbm, eids_hbm, table_hbm, tot_hbm, alive_hbm,
                    nids_v, eids_v, vals_v, ones_v, zero_v,
                    table_s, tot_s, alive_s):
    cid = lax.axis_index("c")
    sid = lax.axis_index("s")
    wid = sid * NC + cid

    def init_body(i, _):
        o = i * L
        zero_v[pl.ds(o, L)] = jnp.zeros((L,), jnp.int32)
        return 0

    lax.fori_loop(0, SLICE // L, init_body, 0)

    def ones_body(i, _):
        o = i * L
        ones_v[pl.ds(o, L)] = jnp.ones((L,), jnp.int32)
        return 0

    lax.fori_loop(0, CHUNK // L, ones_body, 0)

    sslice = pl.multiple_of(sid * SLICE, 8)
    pltpu.sync_copy(zero_v, tot_s.at[pl.ds(sslice, SLICE)])
    pltpu.sync_copy(zero_v, alive_s.at[pl.ds(sslice, SLICE)])
    pltpu.sync_copy(table_hbm.at[pl.ds(sslice, SLICE)],
                    table_s.at[pl.ds(sslice, SLICE)])
    plsc.subcore_barrier()

    base = wid * C_PER_W

    def chunk_body(k, _):
        off = pl.multiple_of(base + k * CHUNK, 8)
        pltpu.sync_copy(nids_hbm.at[pl.ds(off, CHUNK)], nids_v)
        pltpu.sync_copy(eids_hbm.at[pl.ds(off, CHUNK)], eids_v)
        pltpu.sync_copy(table_s.at[eids_v], vals_v)
        pltpu.sync_copy(vals_v, alive_s.at[nids_v], add=True)
        pltpu.sync_copy(ones_v, tot_s.at[nids_v], add=True)
        return 0

    lax.fori_loop(0, N_CHUNKS, chunk_body, 0)
    plsc.subcore_barrier()

    out_off = pl.multiple_of(cid * P + sslice, 8)
    pltpu.sync_copy(tot_s.at[pl.ds(sslice, SLICE)], tot_hbm.at[pl.ds(out_off, SLICE)])
    pltpu.sync_copy(alive_s.at[pl.ds(sslice, SLICE)], alive_hbm.at[pl.ds(out_off, SLICE)])


@functools.partial(
    pl.kernel,
    out_type=jax.ShapeDtypeStruct((P,), jnp.int32),
    mesh=_mesh,
    compiler_params=pltpu.CompilerParams(needs_layout_passes=False),
    scratch_types=[
        pltpu.VMEM((E_PER_W,), jnp.int32),
        pltpu.VMEM((E_PER_W,), jnp.int32),
        pltpu.VMEM((E_PER_W,), jnp.int32),
        pltpu.VMEM((E_PER_W,), jnp.int32),
        pltpu.VMEM((E_PER_W,), jnp.int32),
        pltpu.VMEM((E_PER_W,), jnp.int32),
    ],
)
def _neuron_kernel(tot_hbm, alive_hbm, nm_hbm, out_hbm,
                   t0_v, t1_v, a0_v, a1_v, nm_v, out_v):
    base = pl.multiple_of(_wid() * E_PER_W, 8)
    pltpu.sync_copy(tot_hbm.at[pl.ds(base, E_PER_W)], t0_v)
    pltpu.sync_copy(tot_hbm.at[pl.ds(P + base, E_PER_W)], t1_v)
    pltpu.sync_copy(alive_hbm.at[pl.ds(base, E_PER_W)], a0_v)
    pltpu.sync_copy(alive_hbm.at[pl.ds(P + base, E_PER_W)], a1_v)
    pltpu.sync_copy(nm_hbm.at[pl.ds(base, E_PER_W)], nm_v)

    def body(i, _):
        o = i * L
        tot = t0_v[pl.ds(o, L)] + t1_v[pl.ds(o, L)]
        alv = a0_v[pl.ds(o, L)] + a1_v[pl.ds(o, L)]
        nm = nm_v[pl.ds(o, L)]
        has = tot > 0
        totf = tot.astype(jnp.float32)
        alvf = alv.astype(jnp.float32)
        safe = jnp.where(has, totf, 1.0)
        dr = jnp.where(has, 1.0 - alvf / safe, 0.0)
        apop = (dr > DEAD_RATIO) & (nm != 0)
        out_v[pl.ds(o, L)] = jnp.where(apop, 0, nm)
        return 0

    lax.fori_loop(0, E_PER_W // L, body, 0)
    pltpu.sync_copy(out_v, out_hbm.at[pl.ds(base, E_PER_W)])


def kernel(vfe_masked, VFE_full, hyperedge_index, task_importance_mask,
           neuron_mask, edge_mask, contribution_history, history_idx,
           low_contrib_count):
    pad_e = P - N_EDGES
    vfe_p = jnp.pad(vfe_masked, (0, pad_e))
    task_p = jnp.pad(task_importance_mask, (0, pad_e)).astype(jnp.int32)
    em_p = jnp.pad(edge_mask, (0, pad_e)).astype(jnp.int32)
    lcc_p = jnp.pad(low_contrib_count, (0, pad_e))
    nm_p = jnp.pad(neuron_mask, (0, P - N_NEURONS)).astype(jnp.int32)
    vfull = jnp.full((L,), VFE_full, jnp.float32)

    em_new_p = _edge_kernel(vfe_p, vfull, task_p, em_p, lcc_p)

    neuron_ids = hyperedge_index[0]
    edge_ids = hyperedge_index[1]
    tot2, alive2 = _scatter_kernel(neuron_ids, edge_ids, em_new_p)
    nm_new_p = _neuron_kernel(tot2, alive2, nm_p)

    edge_mask_new = em_new_p[:N_EDGES] != 0
    nm_kernel = nm_new_p[:N_NEURONS] != 0

    def _apply_protection(args):
        nm_in, nm_out = args
        valid = (neuron_ids < N_NEURONS) & (edge_ids < N_EDGES)
        validf = valid.astype(jnp.float32)
        edge_protected = task_importance_mask[edge_ids].astype(jnp.float32) * validf
        protected = jnp.zeros((N_NEURONS,), jnp.float32).at[neuron_ids].add(edge_protected)
        apop = nm_in & (~nm_out) & (protected == 0)
        return nm_in & (~apop)

    nm_new = lax.cond(
        jnp.any(task_importance_mask), _apply_protection, lambda a: a[1],
        (neuron_mask, nm_kernel))

    active_ratio = nm_new.astype(jnp.float32).mean()
    num_dead = (~nm_new).astype(jnp.int32).sum()
    grow_pred = (active_ratio < 0.8) & (VFE_full > VFE_RATIO) & (num_dead > 0)

    def _grow(operands):
        nm, em = operands
        d = ~nm
        ranks = jnp.cumsum(d.astype(jnp.int32)) - 1
        nd = d.astype(jnp.int32).sum()
        num_to_grow = jnp.minimum(nd, max(1, int(N_NEURONS * GROW_CAP)))
        revive_mask = d & (ranks < num_to_grow)
        nm_new2 = nm | revive_mask

        def body(ni, em_cur):
            do = revive_mask[ni]
            dead_occ = (neuron_ids == ni) & (~em_cur[edge_ids]) & do
            cnt = dead_occ.astype(jnp.int32).sum()
            n_rev = jnp.maximum(1, cnt // 2)
            occ_rank = jnp.cumsum(dead_occ.astype(jnp.int32)) - 1
            set_occ = dead_occ & (occ_rank < n_rev)
            hits = jnp.zeros((N_EDGES,), jnp.int32).at[edge_ids].add(set_occ.astype(jnp.int32))
            return em_cur | (hits > 0)

        em_new2 = jax.lax.fori_loop(0, N_NEURONS, body, em)
        return nm_new2, em_new2

    nm_final, em_final = lax.cond(
        grow_pred, _grow, lambda o: o, (nm_new, edge_mask_new))
    return (nm_final, em_final)


# speedup vs baseline: 377.6789x; 1.1148x over previous
import functools

import jax
import jax.numpy as jnp
from jax import lax
from jax.experimental import pallas as pl
from jax.experimental.pallas import tpu as pltpu
from jax.experimental.pallas import tpu_sc as plsc

N_NEURONS = 100000
N_EDGES = 100000
N_CONN = 3200000
COOLDOWN = 10
DEAD_RATIO = 0.9
VFE_RATIO = 1.5
GROW_CAP = 0.05

NC = 2
NS = 16
NW = NC * NS
L = 16

P = 102400
Q = P // 4
Q_PER_W = Q // NW
E_PER_W = P // NW
C_PER_W = N_CONN // NW
CHUNK = 10000
N_CHUNKS = C_PER_W // CHUNK
SLICE = P // NS

_mesh = plsc.VectorSubcoreMesh(
    core_axis_name="c", subcore_axis_name="s", num_cores=NC, num_subcores=NS)


def _wid():
    return lax.axis_index("s") * NC + lax.axis_index("c")


@functools.partial(
    pl.kernel,
    out_type=[
        jax.ShapeDtypeStruct((P,), jnp.int32),
        jax.ShapeDtypeStruct((Q,), jnp.int32),
    ],
    mesh=_mesh,
    compiler_params=pltpu.CompilerParams(needs_layout_passes=False),
    scratch_types=[
        pltpu.VMEM((Q_PER_W,), jnp.float32),
        pltpu.VMEM((L,), jnp.float32),
        pltpu.VMEM((Q_PER_W,), jnp.int32),
        pltpu.VMEM((Q_PER_W,), jnp.int32),
        pltpu.VMEM((Q_PER_W,), jnp.int32),
        pltpu.VMEM((Q_PER_W,), jnp.int32),
        pltpu.VMEM((Q_PER_W,), jnp.int32),
        pltpu.VMEM((Q_PER_W,), jnp.int32),
        pltpu.VMEM((Q_PER_W,), jnp.int32),
        pltpu.VMEM((Q_PER_W,), jnp.int32),
    ],
)
def _edge_kernel(vfe_hbm, vfull_hbm, task_hbm, em_hbm, lcc_hbm,
                 out_hbm, pk_hbm,
                 vfe_v, vfull_v, task_v, em_v, lcc_v,
                 new0_v, new1_v, new2_v, new3_v, pk_v):
    new_bufs = (new0_v, new1_v, new2_v, new3_v)
    wid = _wid()
    pltpu.sync_copy(vfull_hbm, vfull_v)
    vfull = vfull_v[...]

    for q in range(4):
        base = pl.multiple_of(q * Q + wid * Q_PER_W, 8)
        pltpu.sync_copy(vfe_hbm.at[pl.ds(base, Q_PER_W)], vfe_v)
        pltpu.sync_copy(task_hbm.at[pl.ds(base, Q_PER_W)], task_v)
        pltpu.sync_copy(em_hbm.at[pl.ds(base, Q_PER_W)], em_v)
        pltpu.sync_copy(lcc_hbm.at[pl.ds(base, Q_PER_W)], lcc_v)

        def body(i, _):
            o = i * L
            vfe = vfe_v[pl.ds(o, L)]
            task = task_v[pl.ds(o, L)]
            em = em_v[pl.ds(o, L)]
            lcc = lcc_v[pl.ds(o, L)]
            contribution = vfe - vfull
            is_low = contribution <= 0.0
            lcc2 = jnp.where(is_low, lcc + 1, 0)
            apop = (lcc2 >= COOLDOWN) & (task == 0) & (em != 0)
            new_bufs[q][pl.ds(o, L)] = jnp.where(apop, 0, em)
            return 0

        lax.fori_loop(0, Q_PER_W // L, body, 0)
        pltpu.sync_copy(new_bufs[q], out_hbm.at[pl.ds(base, Q_PER_W)])

    def pack_body(i, _):
        o = i * L
        pk = (new0_v[pl.ds(o, L)]
              | (new1_v[pl.ds(o, L)] << 8)
              | (new2_v[pl.ds(o, L)] << 16)
              | (new3_v[pl.ds(o, L)] << 24))
        pk_v[pl.ds(o, L)] = pk
        return 0

    lax.fori_loop(0, Q_PER_W // L, pack_body, 0)
    pk_base = pl.multiple_of(wid * Q_PER_W, 8)
    pltpu.sync_copy(pk_v, pk_hbm.at[pl.ds(pk_base, Q_PER_W)])


@functools.partial(
    pl.kernel,
    out_type=[
        jax.ShapeDtypeStruct((NC * P,), jnp.int32),
        jax.ShapeDtypeStruct((NC * P,), jnp.int32),
    ],
    mesh=_mesh,
    compiler_params=pltpu.CompilerParams(needs_layout_passes=False),
    scratch_types=[
        pltpu.VMEM((Q,), jnp.int32),
        pltpu.VMEM((CHUNK,), jnp.int32),
        pltpu.VMEM((CHUNK,), jnp.int32),
        pltpu.VMEM((CHUNK,), jnp.int32),
        pltpu.VMEM((CHUNK,), jnp.int32),
        pltpu.VMEM((SLICE,), jnp.int32),
        pltpu.VMEM_SHARED((P,), jnp.int32),
        pltpu.VMEM_SHARED((P,), jnp.int32),
    ],
)
def _scatter_kernel(nids_hbm, eids_hbm, ptable_hbm, tot_hbm, dead_hbm,
                    ptable_v, nids_v, eids_v, vals_v, ones_v, zero_v,
                    tot_s, dead_s):
    cid = lax.axis_index("c")
    sid = lax.axis_index("s")
    wid = sid * NC + cid

    def init_body(i, _):
        o = i * L
        zero_v[pl.ds(o, L)] = jnp.zeros((L,), jnp.int32)
        return 0

    lax.fori_loop(0, SLICE // L, init_body, 0)

    def ones_body(i, _):
        o = i * L
        ones_v[pl.ds(o, L)] = jnp.ones((L,), jnp.int32)
        return 0

    lax.fori_loop(0, CHUNK // L, ones_body, 0)

    sslice = pl.multiple_of(sid * SLICE, 8)
    pltpu.sync_copy(zero_v, tot_s.at[pl.ds(sslice, SLICE)])
    pltpu.sync_copy(zero_v, dead_s.at[pl.ds(sslice, SLICE)])
    pltpu.sync_copy(ptable_hbm, ptable_v)
    plsc.subcore_barrier()

    base = wid * C_PER_W

    def chunk_body(k, _):
        off = pl.multiple_of(base + k * CHUNK, 8)
        pltpu.sync_copy(nids_hbm.at[pl.ds(off, CHUNK)], nids_v)
        pltpu.sync_copy(eids_hbm.at[pl.ds(off, CHUNK)], eids_v)

        def gather_body(j, acc):
            o = j * L
            ev = eids_v[pl.ds(o, L)]
            b = ((ev >= Q).astype(jnp.int32)
                 + (ev >= 2 * Q).astype(jnp.int32)
                 + (ev >= 3 * Q).astype(jnp.int32))
            w = ev - b * Q
            pw = plsc.load_gather(ptable_v, [w])
            alive = (pw >> (b << 3)) & 1
            dead = alive ^ 1
            vals_v[pl.ds(o, L)] = dead
            return acc | dead

        dead_acc = lax.fori_loop(
            0, CHUNK // L, gather_body, jnp.zeros((L,), jnp.int32))
        any_dead = jnp.max(dead_acc)

        pltpu.sync_copy(ones_v, tot_s.at[nids_v], add=True)

        @pl.when(any_dead > 0)
        def _():
            pltpu.sync_copy(vals_v, dead_s.at[nids_v], add=True)

        return 0

    lax.fori_loop(0, N_CHUNKS, chunk_body, 0)
    plsc.subcore_barrier()

    out_off = pl.multiple_of(cid * P + sslice, 8)
    pltpu.sync_copy(tot_s.at[pl.ds(sslice, SLICE)], tot_hbm.at[pl.ds(out_off, SLICE)])
    pltpu.sync_copy(dead_s.at[pl.ds(sslice, SLICE)], dead_hbm.at[pl.ds(out_off, SLICE)])


@functools.partial(
    pl.kernel,
    out_type=jax.ShapeDtypeStruct((P,), jnp.int32),
    mesh=_mesh,
    compiler_params=pltpu.CompilerParams(needs_layout_passes=False),
    scratch_types=[
        pltpu.VMEM((E_PER_W,), jnp.int32),
        pltpu.VMEM((E_PER_W,), jnp.int32),
        pltpu.VMEM((E_PER_W,), jnp.int32),
        pltpu.VMEM((E_PER_W,), jnp.int32),
        pltpu.VMEM((E_PER_W,), jnp.int32),
        pltpu.VMEM((E_PER_W,), jnp.int32),
    ],
)
def _neuron_kernel(tot_hbm, dead_hbm, nm_hbm, out_hbm,
                   t0_v, t1_v, d0_v, d1_v, nm_v, out_v):
    base = pl.multiple_of(_wid() * E_PER_W, 8)
    pltpu.sync_copy(tot_hbm.at[pl.ds(base, E_PER_W)], t0_v)
    pltpu.sync_copy(tot_hbm.at[pl.ds(P + base, E_PER_W)], t1_v)
    pltpu.sync_copy(dead_hbm.at[pl.ds(base, E_PER_W)], d0_v)
    pltpu.sync_copy(dead_hbm.at[pl.ds(P + base, E_PER_W)], d1_v)
    pltpu.sync_copy(nm_hbm.at[pl.ds(base, E_PER_W)], nm_v)

    def body(i, _):
        o = i * L
        tot = t0_v[pl.ds(o, L)] + t1_v[pl.ds(o, L)]
        dead = d0_v[pl.ds(o, L)] + d1_v[pl.ds(o, L)]
        alv = tot - dead
        nm = nm_v[pl.ds(o, L)]
        has = tot > 0
        totf = tot.astype(jnp.float32)
        alvf = alv.astype(jnp.float32)
        safe = jnp.where(has, totf, 1.0)
        dr = jnp.where(has, 1.0 - alvf / safe, 0.0)
        apop = (dr > DEAD_RATIO) & (nm != 0)
        out_v[pl.ds(o, L)] = jnp.where(apop, 0, nm)
        return 0

    lax.fori_loop(0, E_PER_W // L, body, 0)
    pltpu.sync_copy(out_v, out_hbm.at[pl.ds(base, E_PER_W)])


def kernel(vfe_masked, VFE_full, hyperedge_index, task_importance_mask,
           neuron_mask, edge_mask, contribution_history, history_idx,
           low_contrib_count):
    pad_e = P - N_EDGES
    vfe_p = jnp.pad(vfe_masked, (0, pad_e))
    task_p = jnp.pad(task_importance_mask, (0, pad_e)).astype(jnp.int32)
    em_p = jnp.pad(edge_mask, (0, pad_e)).astype(jnp.int32)
    lcc_p = jnp.pad(low_contrib_count, (0, pad_e))
    nm_p = jnp.pad(neuron_mask, (0, P - N_NEURONS)).astype(jnp.int32)
    vfull = jnp.full((L,), VFE_full, jnp.float32)

    em_new_p, ptable = _edge_kernel(vfe_p, vfull, task_p, em_p, lcc_p)

    neuron_ids = hyperedge_index[0]
    edge_ids = hyperedge_index[1]
    tot2, dead2 = _scatter_kernel(neuron_ids, edge_ids, ptable)
    nm_new_p = _neuron_kernel(tot2, dead2, nm_p)

    edge_mask_new = em_new_p[:N_EDGES] != 0
    nm_kernel = nm_new_p[:N_NEURONS] != 0

    def _apply_protection(args):
        nm_in, nm_out = args
        valid = (neuron_ids < N_NEURONS) & (edge_ids < N_EDGES)
        validf = valid.astype(jnp.float32)
        edge_protected = task_importance_mask[edge_ids].astype(jnp.float32) * validf
        protected = jnp.zeros((N_NEURONS,), jnp.float32).at[neuron_ids].add(edge_protected)
        apop = nm_in & (~nm_out) & (protected == 0)
        return nm_in & (~apop)

    nm_new = lax.cond(
        jnp.any(task_importance_mask), _apply_protection, lambda a: a[1],
        (neuron_mask, nm_kernel))

    active_ratio = nm_new.astype(jnp.float32).mean()
    num_dead = (~nm_new).astype(jnp.int32).sum()
    grow_pred = (active_ratio < 0.8) & (VFE_full > VFE_RATIO) & (num_dead > 0)

    def _grow(operands):
        nm, em = operands
        d = ~nm
        ranks = jnp.cumsum(d.astype(jnp.int32)) - 1
        nd = d.astype(jnp.int32).sum()
        num_to_grow = jnp.minimum(nd, max(1, int(N_NEURONS * GROW_CAP)))
        revive_mask = d & (ranks < num_to_grow)
        nm_new2 = nm | revive_mask

        def body(ni, em_cur):
            do = revive_mask[ni]
            dead_occ = (neuron_ids == ni) & (~em_cur[edge_ids]) & do
            cnt = dead_occ.astype(jnp.int32).sum()
            n_rev = jnp.maximum(1, cnt // 2)
            occ_rank = jnp.cumsum(dead_occ.astype(jnp.int32)) - 1
            set_occ = dead_occ & (occ_rank < n_rev)
            hits = jnp.zeros((N_EDGES,), jnp.int32).at[edge_ids].add(set_occ.astype(jnp.int32))
            return em_cur | (hits > 0)

        em_new2 = jax.lax.fori_loop(0, N_NEURONS, body, em)
        return nm_new2, em_new2

    nm_final, em_final = lax.cond(
        grow_pred, _grow, lambda o: o, (nm_new, edge_mask_new))
    return (nm_final, em_final)


# speedup vs baseline: 396.5732x; 1.0500x over previous
import functools

import jax
import jax.numpy as jnp
from jax import lax
from jax.experimental import pallas as pl
from jax.experimental.pallas import tpu as pltpu
from jax.experimental.pallas import tpu_sc as plsc

N_NEURONS = 100000
N_EDGES = 100000
N_CONN = 3200000
COOLDOWN = 10
DEAD_RATIO = 0.9
VFE_RATIO = 1.5
GROW_CAP = 0.05

NC = 2
NS = 16
NW = NC * NS
L = 16

P = 102400
Q = P // 4
Q_PER_W = Q // NW
E_PER_W = P // NW
C_PER_W = N_CONN // NW
CHUNK = 10000
N_CHUNKS = C_PER_W // CHUNK
SLICE = P // NS

_mesh = plsc.VectorSubcoreMesh(
    core_axis_name="c", subcore_axis_name="s", num_cores=NC, num_subcores=NS)


def _wid():
    return lax.axis_index("s") * NC + lax.axis_index("c")


@functools.partial(
    pl.kernel,
    out_type=[
        jax.ShapeDtypeStruct((P,), jnp.int32),
        jax.ShapeDtypeStruct((Q,), jnp.int32),
        jax.ShapeDtypeStruct((NW * L,), jnp.int32),
    ],
    mesh=_mesh,
    compiler_params=pltpu.CompilerParams(needs_layout_passes=False),
    scratch_types=(
        [pltpu.VMEM((L,), jnp.float32)]
        + [pltpu.VMEM((Q_PER_W,), jnp.float32) for _ in range(4)]
        + [pltpu.VMEM((Q_PER_W,), jnp.int32) for _ in range(12)]
        + [pltpu.VMEM((Q_PER_W,), jnp.int32) for _ in range(4)]
        + [pltpu.VMEM((Q_PER_W,), jnp.int32)]
        + [pltpu.VMEM((L,), jnp.int32)]
        + [pltpu.SemaphoreType.DMA]
    ),
)
def _edge_kernel(vfe_hbm, vfull_hbm, task_hbm, em_hbm, lcc_hbm,
                 out_hbm, pk_hbm, flag_hbm, *scratch):
    vfull_v = scratch[0]
    vfe_b = scratch[1:5]
    task_b = scratch[5:9]
    em_b = scratch[9:13]
    lcc_b = scratch[13:17]
    new_b = scratch[17:21]
    pk_v = scratch[21]
    flag_v = scratch[22]
    sem = scratch[23]

    wid = _wid()
    descs = [pltpu.async_copy(vfull_hbm, vfull_v, sem)]
    bases = []
    for q in range(4):
        base = pl.multiple_of(q * Q + wid * Q_PER_W, 8)
        bases.append(base)
        descs.append(pltpu.async_copy(vfe_hbm.at[pl.ds(base, Q_PER_W)], vfe_b[q], sem))
        descs.append(pltpu.async_copy(task_hbm.at[pl.ds(base, Q_PER_W)], task_b[q], sem))
        descs.append(pltpu.async_copy(em_hbm.at[pl.ds(base, Q_PER_W)], em_b[q], sem))
        descs.append(pltpu.async_copy(lcc_hbm.at[pl.ds(base, Q_PER_W)], lcc_b[q], sem))
    for d in descs:
        d.wait()

    vfull = vfull_v[...]
    for q in range(4):

        def body(i, facc):
            o = i * L
            vfe = vfe_b[q][pl.ds(o, L)]
            task = task_b[q][pl.ds(o, L)]
            em = em_b[q][pl.ds(o, L)]
            lcc = lcc_b[q][pl.ds(o, L)]
            contribution = vfe - vfull
            is_low = contribution <= 0.0
            lcc2 = jnp.where(is_low, lcc + 1, 0)
            apop = (lcc2 >= COOLDOWN) & (task == 0) & (em != 0)
            new = jnp.where(apop, 0, em)
            new_b[q][pl.ds(o, L)] = new
            return facc | (new == 0).astype(jnp.int32)

        facc_q = lax.fori_loop(0, Q_PER_W // L, body, jnp.zeros((L,), jnp.int32))
        flag_v[...] = (flag_v[...] | facc_q) if q else facc_q

    def pack_body(i, _):
        o = i * L
        pk = (new_b[0][pl.ds(o, L)]
              | (new_b[1][pl.ds(o, L)] << 8)
              | (new_b[2][pl.ds(o, L)] << 16)
              | (new_b[3][pl.ds(o, L)] << 24))
        pk_v[pl.ds(o, L)] = pk
        return 0

    lax.fori_loop(0, Q_PER_W // L, pack_body, 0)

    out_descs = []
    for q in range(4):
        out_descs.append(pltpu.async_copy(
            new_b[q], out_hbm.at[pl.ds(bases[q], Q_PER_W)], sem))
    pk_base = pl.multiple_of(wid * Q_PER_W, 8)
    out_descs.append(pltpu.async_copy(pk_v, pk_hbm.at[pl.ds(pk_base, Q_PER_W)], sem))
    fl_base = pl.multiple_of(wid * L, 8)
    out_descs.append(pltpu.async_copy(flag_v, flag_hbm.at[pl.ds(fl_base, L)], sem))
    for d in out_descs:
        d.wait()


@functools.partial(
    pl.kernel,
    out_type=[
        jax.ShapeDtypeStruct((NC * P,), jnp.int32),
        jax.ShapeDtypeStruct((NC * P,), jnp.int32),
    ],
    mesh=_mesh,
    compiler_params=pltpu.CompilerParams(needs_layout_passes=False),
    scratch_types=[
        pltpu.VMEM((Q,), jnp.int32),
        pltpu.VMEM((CHUNK,), jnp.int32),
        pltpu.VMEM((CHUNK,), jnp.int32),
        pltpu.VMEM((CHUNK,), jnp.int32),
        pltpu.VMEM((CHUNK,), jnp.int32),
        pltpu.VMEM((CHUNK,), jnp.int32),
        pltpu.VMEM((SLICE,), jnp.int32),
        pltpu.VMEM((NW * L,), jnp.int32),
        pltpu.VMEM_SHARED((P,), jnp.int32),
        pltpu.VMEM_SHARED((P,), jnp.int32),
        pltpu.SemaphoreType.DMA,
    ],
)
def _scatter_kernel(nids_hbm, eids_hbm, ptable_hbm, flag_hbm, tot_hbm, dead_hbm,
                    ptable_v, nids0_v, nids1_v, eids_v, vals_v, ones_v, zero_v,
                    flags_v, tot_s, dead_s, lsem):
    cid = lax.axis_index("c")
    sid = lax.axis_index("s")
    wid = sid * NC + cid
    nbuf = (nids0_v, nids1_v)

    def init_body(i, _):
        o = i * L
        zero_v[pl.ds(o, L)] = jnp.zeros((L,), jnp.int32)
        return 0

    lax.fori_loop(0, SLICE // L, init_body, 0)

    def ones_body(i, _):
        o = i * L
        ones_v[pl.ds(o, L)] = jnp.ones((L,), jnp.int32)
        return 0

    lax.fori_loop(0, CHUNK // L, ones_body, 0)

    pltpu.sync_copy(flag_hbm, flags_v)

    def flag_body(i, acc):
        return acc | flags_v[pl.ds(i * L, L)]

    flag_acc = lax.fori_loop(0, NW, flag_body, jnp.zeros((L,), jnp.int32))
    have_dead = jnp.max(flag_acc) > 0

    sslice = pl.multiple_of(sid * SLICE, 8)
    pltpu.sync_copy(zero_v, tot_s.at[pl.ds(sslice, SLICE)])
    pltpu.sync_copy(zero_v, dead_s.at[pl.ds(sslice, SLICE)])

    @pl.when(have_dead)
    def _():
        pltpu.sync_copy(ptable_hbm, ptable_v)

    plsc.subcore_barrier()

    base = wid * C_PER_W
    pltpu.async_copy(nids_hbm.at[pl.ds(pl.multiple_of(base, 8), CHUNK)],
                     nbuf[0], lsem)

    def outer_body(k2, _):
        for b in range(2):
            k = k2 * 2 + b
            pltpu.make_async_copy(
                nids_hbm.at[pl.ds(0, CHUNK)], nbuf[b], lsem).wait()

            @pl.when(k < N_CHUNKS - 1)
            def _():
                off2 = pl.multiple_of(base + (k + 1) * CHUNK, 8)
                pltpu.async_copy(nids_hbm.at[pl.ds(off2, CHUNK)],
                                 nbuf[1 - b], lsem)

            @pl.when(have_dead)
            def _():
                off = pl.multiple_of(base + k * CHUNK, 8)
                pltpu.sync_copy(eids_hbm.at[pl.ds(off, CHUNK)], eids_v)

                def gather_body(j, acc):
                    o = j * L
                    ev = eids_v[pl.ds(o, L)]
                    bq = ((ev >= Q).astype(jnp.int32)
                          + (ev >= 2 * Q).astype(jnp.int32)
                          + (ev >= 3 * Q).astype(jnp.int32))
                    w = ev - bq * Q
                    pw = plsc.load_gather(ptable_v, [w])
                    dead = ((pw >> (bq << 3)) & 1) ^ 1
                    vals_v[pl.ds(o, L)] = dead
                    return acc | dead

                dead_acc = lax.fori_loop(
                    0, CHUNK // L, gather_body, jnp.zeros((L,), jnp.int32))

                @pl.when(jnp.max(dead_acc) > 0)
                def _():
                    pltpu.sync_copy(vals_v, dead_s.at[nbuf[b]], add=True)

            pltpu.sync_copy(ones_v, tot_s.at[nbuf[b]], add=True)
        return 0

    lax.fori_loop(0, N_CHUNKS // 2, outer_body, 0)
    plsc.subcore_barrier()

    out_off = pl.multiple_of(cid * P + sslice, 8)
    pltpu.sync_copy(tot_s.at[pl.ds(sslice, SLICE)], tot_hbm.at[pl.ds(out_off, SLICE)])
    pltpu.sync_copy(dead_s.at[pl.ds(sslice, SLICE)], dead_hbm.at[pl.ds(out_off, SLICE)])


@functools.partial(
    pl.kernel,
    out_type=jax.ShapeDtypeStruct((P,), jnp.int32),
    mesh=_mesh,
    compiler_params=pltpu.CompilerParams(needs_layout_passes=False),
    scratch_types=[
        pltpu.VMEM((E_PER_W,), jnp.int32),
        pltpu.VMEM((E_PER_W,), jnp.int32),
        pltpu.VMEM((E_PER_W,), jnp.int32),
        pltpu.VMEM((E_PER_W,), jnp.int32),
        pltpu.VMEM((E_PER_W,), jnp.int32),
        pltpu.VMEM((E_PER_W,), jnp.int32),
        pltpu.SemaphoreType.DMA,
    ],
)
def _neuron_kernel(tot_hbm, dead_hbm, nm_hbm, out_hbm,
                   t0_v, t1_v, d0_v, d1_v, nm_v, out_v, sem):
    base = pl.multiple_of(_wid() * E_PER_W, 8)
    descs = [
        pltpu.async_copy(tot_hbm.at[pl.ds(base, E_PER_W)], t0_v, sem),
        pltpu.async_copy(tot_hbm.at[pl.ds(P + base, E_PER_W)], t1_v, sem),
        pltpu.async_copy(dead_hbm.at[pl.ds(base, E_PER_W)], d0_v, sem),
        pltpu.async_copy(dead_hbm.at[pl.ds(P + base, E_PER_W)], d1_v, sem),
        pltpu.async_copy(nm_hbm.at[pl.ds(base, E_PER_W)], nm_v, sem),
    ]
    for d in descs:
        d.wait()

    def body(i, _):
        o = i * L
        tot = t0_v[pl.ds(o, L)] + t1_v[pl.ds(o, L)]
        dead = d0_v[pl.ds(o, L)] + d1_v[pl.ds(o, L)]
        alv = tot - dead
        nm = nm_v[pl.ds(o, L)]
        has = tot > 0
        totf = tot.astype(jnp.float32)
        alvf = alv.astype(jnp.float32)
        safe = jnp.where(has, totf, 1.0)
        dr = jnp.where(has, 1.0 - alvf / safe, 0.0)
        apop = (dr > DEAD_RATIO) & (nm != 0)
        out_v[pl.ds(o, L)] = jnp.where(apop, 0, nm)
        return 0

    lax.fori_loop(0, E_PER_W // L, body, 0)
    pltpu.sync_copy(out_v, out_hbm.at[pl.ds(base, E_PER_W)])


def kernel(vfe_masked, VFE_full, hyperedge_index, task_importance_mask,
           neuron_mask, edge_mask, contribution_history, history_idx,
           low_contrib_count):
    pad_e = P - N_EDGES
    vfe_p = jnp.pad(vfe_masked, (0, pad_e))
    task_p = jnp.pad(task_importance_mask, (0, pad_e)).astype(jnp.int32)
    em_p = jnp.pad(edge_mask, (0, pad_e)).astype(jnp.int32)
    lcc_p = jnp.pad(low_contrib_count, (0, pad_e))
    nm_p = jnp.pad(neuron_mask, (0, P - N_NEURONS)).astype(jnp.int32)
    vfull = jnp.full((L,), VFE_full, jnp.float32)

    em_new_p, ptable, dead_flags = _edge_kernel(vfe_p, vfull, task_p, em_p, lcc_p)

    neuron_ids = hyperedge_index[0]
    edge_ids = hyperedge_index[1]
    tot2, dead2 = _scatter_kernel(neuron_ids, edge_ids, ptable, dead_flags)
    nm_new_p = _neuron_kernel(tot2, dead2, nm_p)

    edge_mask_new = em_new_p[:N_EDGES] != 0
    nm_kernel = nm_new_p[:N_NEURONS] != 0

    def _apply_protection(args):
        nm_in, nm_out = args
        valid = (neuron_ids < N_NEURONS) & (edge_ids < N_EDGES)
        validf = valid.astype(jnp.float32)
        edge_protected = task_importance_mask[edge_ids].astype(jnp.float32) * validf
        protected = jnp.zeros((N_NEURONS,), jnp.float32).at[neuron_ids].add(edge_protected)
        apop = nm_in & (~nm_out) & (protected == 0)
        return nm_in & (~apop)

    nm_new = lax.cond(
        jnp.any(task_importance_mask), _apply_protection, lambda a: a[1],
        (neuron_mask, nm_kernel))

    active_ratio = nm_new.astype(jnp.float32).mean()
    num_dead = (~nm_new).astype(jnp.int32).sum()
    grow_pred = (active_ratio < 0.8) & (VFE_full > VFE_RATIO) & (num_dead > 0)

    def _grow(operands):
        nm, em = operands
        d = ~nm
        ranks = jnp.cumsum(d.astype(jnp.int32)) - 1
        nd = d.astype(jnp.int32).sum()
        num_to_grow = jnp.minimum(nd, max(1, int(N_NEURONS * GROW_CAP)))
        revive_mask = d & (ranks < num_to_grow)
        nm_new2 = nm | revive_mask

        def body(ni, em_cur):
            do = revive_mask[ni]
            dead_occ = (neuron_ids == ni) & (~em_cur[edge_ids]) & do
            cnt = dead_occ.astype(jnp.int32).sum()
            n_rev = jnp.maximum(1, cnt // 2)
            occ_rank = jnp.cumsum(dead_occ.astype(jnp.int32)) - 1
            set_occ = dead_occ & (occ_rank < n_rev)
            hits = jnp.zeros((N_EDGES,), jnp.int32).at[edge_ids].add(set_occ.astype(jnp.int32))
            return em_cur | (hits > 0)

        em_new2 = jax.lax.fori_loop(0, N_NEURONS, body, em)
        return nm_new2, em_new2

    nm_final, em_final = lax.cond(
        grow_pred, _grow, lambda o: o, (nm_new, edge_mask_new))
    return (nm_final, em_final)
